# Initial kernel scaffold; baseline (speedup 1.0000x reference)
import jax
import jax.numpy as jnp
from jax.experimental import pallas as pl


def kernel(ids, feats, adj, W_x1, b_x1, W_n1, b_n1, W_x2, b_x2, W_n2, b_n2, W_fc, b_fc):
    raise NotImplementedError("write your pallas kernel here")


# speedup vs baseline: 4.2539x; 4.2539x over previous
import jax
import jax.numpy as jnp
import numpy as np
from jax import lax
from jax.experimental import pallas as pl
from jax.experimental.pallas import tpu as pltpu
from jax.experimental.pallas import tpu_sc as plsc

N = 100000
D = 128
MAXDEG = 32
B = 1024
H = 128
NCLS = 40
NS1 = 25
NS2 = 10

NW = 32
TI = B // NW
T1 = B * NS1 // NW
T2 = T1 * NS2
CH = 80
GC = 40
GR = GC * NS2
MC = T1 // GC
CPM = GR // CH
XR = 2

_AGG = np.repeat(np.eye(64, dtype=np.float32), NS1, axis=1) / NS1


def _sc_body(ids_h, adjf_h, feats_h, f1_h, r2_h, c2_h,
             x0_h, x1_h, m2_h,
             ids_v, f1_v, cur1_v, c2_v, r2_v, g2_v,
             fbuf, mbuf, x0b, sem_a, sem_b):
    t = lax.axis_index("s") * 2 + lax.axis_index("c")
    bi = t * TI
    b1 = t * T1
    b2 = t * T2

    pltpu.sync_copy(ids_h.at[pl.ds(bi, TI)], ids_v)
    pltpu.sync_copy(f1_h.at[pl.ds(b1, T1)], f1_v)
    pltpu.sync_copy(r2_h.at[pl.ds(b2, T2)], r2_v)
    pltpu.sync_copy(c2_h.at[pl.ds(b2, T2)], c2_v)

    pltpu.async_copy(feats_h.at[ids_v], x0b, sem_a).wait()
    pltpu.sync_copy(x0b, x0_h.at[pl.ds(bi, TI)])

    def fire_cur1(j, c):
        pltpu.async_copy(adjf_h.at[f1_v.at[pl.ds(j * CH, CH)]],
                         cur1_v.at[pl.ds(j * CH, CH)], sem_a)
        return c

    lax.fori_loop(0, T1 // CH, fire_cur1, 0)

    def fire_rep(j, c):
        pltpu.async_copy(adjf_h.at[r2_v.at[pl.ds(j * CH, CH)]],
                         g2_v.at[pl.ds(j * CH, CH)], sem_a)
        return c

    lax.fori_loop(0, T2 // CH, fire_rep, 0)

    pltpu.make_async_copy(adjf_h.at[pl.ds(0, T1)], cur1_v, sem_a).wait()
    pltpu.make_async_copy(adjf_h.at[pl.ds(0, T2)], g2_v, sem_a).wait()

    def fidx2_step(m, c):
        o = m * 16
        g2_v[pl.ds(o, 16)] = g2_v[pl.ds(o, 16)] * MAXDEG + c2_v[pl.ds(o, 16)]
        return c

    lax.fori_loop(0, T2 // 16, fidx2_step, 0)

    def fire_cur2(j, c):
        pltpu.async_copy(adjf_h.at[g2_v.at[pl.ds(j * CH, CH)]],
                         r2_v.at[pl.ds(j * CH, CH)], sem_b)
        return c

    lax.fori_loop(0, T2 // CH, fire_cur2, 0)

    def x1_round(r, c):
        def fire(j, cc):
            pltpu.async_copy(
                feats_h.at[cur1_v.at[pl.ds(r * GR + j * CH, CH)]],
                fbuf.at[pl.ds(j * CH, CH)], sem_a)
            return cc

        lax.fori_loop(0, CPM, fire, 0)
        pltpu.make_async_copy(feats_h.at[pl.ds(0, GR)], fbuf, sem_a).wait()
        pltpu.sync_copy(fbuf, x1_h.at[pl.ds(b1 + r * GR, GR)])
        return c

    lax.fori_loop(0, XR, x1_round, 0)

    pltpu.make_async_copy(adjf_h.at[pl.ds(0, T2)], r2_v, sem_b).wait()

    inv_ns2 = jnp.float32(1.0 / NS2)

    def m2_step(mc, carry):
        rbase = mc * GR

        def fire(j, cc):
            pltpu.async_copy(
                feats_h.at[r2_v.at[pl.ds(rbase + j * CH, CH)]],
                fbuf.at[pl.ds(j * CH, CH)], sem_a)
            return cc

        lax.fori_loop(0, CPM, fire, 0)
        pltpu.make_async_copy(feats_h.at[pl.ds(0, GR)], fbuf, sem_a).wait()

        def gstep(g, c2):
            row = g * NS2
            for cb in range(D // 16):
                o = cb * 16
                acc = fbuf[row, pl.ds(o, 16)]
                for j in range(1, NS2):
                    acc = acc + fbuf[row + j, pl.ds(o, 16)]
                mbuf[g, pl.ds(o, 16)] = acc * inv_ns2
            return c2

        lax.fori_loop(0, GC, gstep, 0)
        pltpu.sync_copy(mbuf, m2_h.at[pl.ds(b1 + mc * GC, GC)])
        return carry

    lax.fori_loop(0, MC, m2_step, 0)


_sc_gather = pl.kernel(
    _sc_body,
    out_type=[
        jax.ShapeDtypeStruct((B, D), jnp.float32),
        jax.ShapeDtypeStruct((B * NS1, D), jnp.float32),
        jax.ShapeDtypeStruct((B * NS1, D), jnp.float32),
    ],
    mesh=plsc.VectorSubcoreMesh(core_axis_name="c", subcore_axis_name="s",
                                num_cores=2, num_subcores=16),
    scratch_types=[
        pltpu.VMEM((TI,), jnp.int32),
        pltpu.VMEM((T1,), jnp.int32),
        pltpu.VMEM((T1,), jnp.int32),
        pltpu.VMEM((T2,), jnp.int32),
        pltpu.VMEM((T2,), jnp.int32),
        pltpu.VMEM((T2,), jnp.int32),
        pltpu.VMEM((GR, D), jnp.float32),
        pltpu.VMEM((GC, D), jnp.float32),
        pltpu.VMEM((TI, D), jnp.float32),
        pltpu.SemaphoreType.DMA,
        pltpu.SemaphoreType.DMA,
    ],
)


def _tc1_body(x1_ref, m2_ref, wx_ref, bx_ref, wn_ref, bn_ref, agg_ref,
              m1_ref, ma_ref, mb_ref):
    x1b = x1_ref[...]
    m2b = m2_ref[...]
    a = jnp.maximum(
        jnp.dot(x1b, wx_ref[...], preferred_element_type=jnp.float32)
        + bx_ref[...], 0.0)
    bb = jnp.maximum(
        jnp.dot(m2b, wn_ref[...], preferred_element_type=jnp.float32)
        + bn_ref[...], 0.0)
    agg = agg_ref[...]
    m1_ref[...] = jnp.dot(agg, x1b, preferred_element_type=jnp.float32)
    ma_ref[...] = jnp.dot(agg, a, preferred_element_type=jnp.float32)
    mb_ref[...] = jnp.dot(agg, bb, preferred_element_type=jnp.float32)


_ROWS1 = 1600
_tc1 = pl.pallas_call(
    _tc1_body,
    grid=(B * NS1 // _ROWS1,),
    in_specs=[
        pl.BlockSpec((_ROWS1, D), lambda i: (i, 0)),
        pl.BlockSpec((_ROWS1, D), lambda i: (i, 0)),
        pl.BlockSpec((D, H), lambda i: (0, 0)),
        pl.BlockSpec((1, H), lambda i: (0, 0)),
        pl.BlockSpec((D, H), lambda i: (0, 0)),
        pl.BlockSpec((1, H), lambda i: (0, 0)),
        pl.BlockSpec((64, _ROWS1), lambda i: (0, 0)),
    ],
    out_specs=[
        pl.BlockSpec((64, D), lambda i: (i, 0)),
        pl.BlockSpec((64, H), lambda i: (i, 0)),
        pl.BlockSpec((64, H), lambda i: (i, 0)),
    ],
    out_shape=[
        jax.ShapeDtypeStruct((B, D), jnp.float32),
        jax.ShapeDtypeStruct((B, H), jnp.float32),
        jax.ShapeDtypeStruct((B, H), jnp.float32),
    ],
)


def _tc2_body(x0_ref, m1_ref, ma_ref, mb_ref,
              wx1_ref, bx1_ref, wn1_ref, bn1_ref,
              wx2t_ref, wx2b_ref, bx2_ref,
              wn2t_ref, wn2b_ref, bn2_ref,
              wfct_ref, wfcb_ref, bfc_ref, out_ref):
    f32 = jnp.float32

    def mm(x, w):
        return jnp.dot(x, w, preferred_element_type=f32)

    h0a = jnp.maximum(mm(x0_ref[...], wx1_ref[...]) + bx1_ref[...], 0.0)
    h0b = jnp.maximum(mm(m1_ref[...], wn1_ref[...]) + bn1_ref[...], 0.0)
    ga = jnp.maximum(
        mm(h0a, wx2t_ref[...]) + mm(h0b, wx2b_ref[...]) + bx2_ref[...], 0.0)
    gb = jnp.maximum(
        mm(ma_ref[...], wn2t_ref[...]) + mm(mb_ref[...], wn2b_ref[...])
        + bn2_ref[...], 0.0)
    nsq = (jnp.sum(ga * ga, axis=1, keepdims=True)
           + jnp.sum(gb * gb, axis=1, keepdims=True))
    inv = 1.0 / jnp.maximum(jnp.sqrt(nsq), 1e-12)
    out_ref[...] = (mm(ga * inv, wfct_ref[...]) + mm(gb * inv, wfcb_ref[...])
                    + bfc_ref[...])


_tc2 = pl.pallas_call(
    _tc2_body,
    out_shape=jax.ShapeDtypeStruct((B, NCLS), jnp.float32),
)


def kernel(ids, feats, adj, W_x1, b_x1, W_n1, b_n1, W_x2, b_x2, W_n2, b_n2,
           W_fc, b_fc):
    ids = ids.astype(jnp.int32)
    cols1 = jax.random.randint(jax.random.key(100), (B, NS1), 0,
                               MAXDEG).astype(jnp.int32).reshape(-1)
    cols2 = jax.random.randint(jax.random.key(101), (B * NS1, NS2), 0,
                               MAXDEG).astype(jnp.int32).reshape(-1)

    adjf = adj.reshape(-1)
    fidx1 = jnp.repeat(ids, NS1) * MAXDEG + cols1
    ridx2 = jnp.repeat(fidx1, NS2)

    x0, x1, m2 = _sc_gather(ids, adjf, feats, fidx1, ridx2, cols2)

    m1, ma, mb = _tc1(x1, m2, W_x1, b_x1.reshape(1, H), W_n1,
                      b_n1.reshape(1, H), _AGG)

    out = _tc2(x0, m1, ma, mb,
               W_x1, b_x1.reshape(1, H), W_n1, b_n1.reshape(1, H),
               W_x2[:H], W_x2[H:], b_x2.reshape(1, H),
               W_n2[:H], W_n2[H:], b_n2.reshape(1, H),
               W_fc[:H], W_fc[H:], b_fc.reshape(1, NCLS))
    return out


# speedup vs baseline: 5.8315x; 1.3709x over previous
import jax
import jax.numpy as jnp
import numpy as np
from jax import lax
from jax.experimental import pallas as pl
from jax.experimental.pallas import tpu as pltpu
from jax.experimental.pallas import tpu_sc as plsc

N = 100000
D = 128
MAXDEG = 32
B = 1024
H = 128
NCLS = 40
NS1 = 25
NS2 = 10

NW = 32
TI = B // NW
T1 = B * NS1 // NW
T2 = T1 * NS2
CHE = 80
GC = 16
GR = GC * NS2
MC = T1 // GC
CHR = 40
CPM = GR // CHR
XRND = T1 // GR

_P25 = np.asarray(np.arange(16 * NS1) // NS1, dtype=np.int32)
_P10 = np.asarray(np.arange(8 * NS2) // NS2, dtype=np.int32)
_AGG = np.repeat(np.eye(64, dtype=np.float32), NS1, axis=1) / NS1

_GDN = lax.GatherDimensionNumbers(
    offset_dims=(), collapsed_slice_dims=(0,), start_index_map=(0,))


def _vgather(src, idx):
    return lax.gather(src, idx[:, None], _GDN, (1,),
                      mode=lax.GatherScatterMode.PROMISE_IN_BOUNDS)


def _sc_body(ids_h, adjf_h, feats_h, c1_h, c2_h, p25_h, p10_h,
             x0_h, x1_h, m2_h,
             ids_v, c1_v, f1_v, cur1_v, c2_v, g2_v, cur2_v,
             p25_v, p10_v, fbuf, mbuf, x0b,
             sem_a, sem_b, sem_c, sem_d):
    t = lax.axis_index("s") * 2 + lax.axis_index("c")
    bi = t * TI
    b1 = t * T1
    b2 = t * T2

    pltpu.sync_copy(ids_h.at[pl.ds(bi, TI)], ids_v)
    pltpu.sync_copy(c1_h.at[pl.ds(b1, T1)], c1_v)
    pltpu.sync_copy(c2_h.at[pl.ds(b2, T2)], c2_v)
    pltpu.sync_copy(p25_h, p25_v)
    pltpu.sync_copy(p10_h, p10_v)

    pltpu.async_copy(feats_h.at[ids_v], x0b, sem_a).wait()
    pltpu.sync_copy(x0b, x0_h.at[pl.ds(bi, TI)])

    for m_st in range(2):
        src = ids_v[pl.ds(m_st * 16, 16)]

        def f1_step(j, c, src=src, m_st=m_st):
            o = m_st * 16 * NS1 + j * 16
            rep = _vgather(src, p25_v[pl.ds(j * 16, 16)])
            f1_v[pl.ds(o, 16)] = rep * MAXDEG + c1_v[pl.ds(o, 16)]
            return c

        lax.fori_loop(0, NS1, f1_step, 0)

    def fire_cur1(j, c):
        pltpu.async_copy(adjf_h.at[f1_v.at[pl.ds(j * CHE, CHE)]],
                         cur1_v.at[pl.ds(j * CHE, CHE)], sem_a)
        return c

    lax.fori_loop(0, T1 // CHE, fire_cur1, 0)
    pltpu.make_async_copy(adjf_h.at[pl.ds(0, T1)],
                          cur1_v.at[pl.ds(0, T1)], sem_a).wait()

    def f2_step(m, c):
        src = cur1_v[pl.ds(m * 8, 16)]
        for j in range(5):
            o = m * 80 + j * 16
            rep = _vgather(src, p10_v[pl.ds(j * 16, 16)])
            g2_v[pl.ds(o, 16)] = rep * MAXDEG + c2_v[pl.ds(o, 16)]
        return c

    lax.fori_loop(0, T2 // 80, f2_step, 0)

    def fire_cur2(j, c):
        pltpu.async_copy(adjf_h.at[g2_v.at[pl.ds(j * CHE, CHE)]],
                         cur2_v.at[pl.ds(j * CHE, CHE)], sem_d)
        return c

    lax.fori_loop(0, T2 // CHE, fire_cur2, 0)

    def x1_fire(r, sem):
        def fire(j, c, r=r):
            pltpu.async_copy(
                feats_h.at[cur1_v.at[pl.ds(r * GR + j * CHR, CHR)]],
                fbuf.at[pl.ds((r % 2) * GR + j * CHR, CHR)], sem)
            return c

        lax.fori_loop(0, CPM, fire, 0)

    def x1_wait(sem):
        pltpu.make_async_copy(feats_h.at[pl.ds(0, GR)],
                              fbuf.at[pl.ds(0, GR)], sem).wait()

    def x1_out(r):
        pltpu.sync_copy(fbuf.at[pl.ds((r % 2) * GR, GR)],
                        x1_h.at[pl.ds(b1 + r * GR, GR)])

    x1_sems = [sem_a, sem_b]
    x1_fire(0, x1_sems[0])
    x1_fire(1, x1_sems[1])
    for r in range(XRND):
        x1_wait(x1_sems[r % 2])
        x1_out(r)
        if r + 2 < XRND:
            x1_fire(r + 2, x1_sems[r % 2])

    pltpu.make_async_copy(adjf_h.at[pl.ds(0, T2)], cur2_v, sem_d).wait()

    inv_ns2 = jnp.float32(1.0 / NS2)

    def m2_fire(c, sem, half):
        def fire(j, cc, c=c, half=half):
            pltpu.async_copy(
                feats_h.at[cur2_v.at[pl.ds(c * GR + j * CHR, CHR)]],
                fbuf.at[pl.ds(half * GR + j * CHR, CHR)], sem)
            return cc

        lax.fori_loop(0, CPM, fire, 0)

    def m2_waitg(sem):
        pltpu.make_async_copy(feats_h.at[pl.ds(0, GR)],
                              fbuf.at[pl.ds(0, GR)], sem).wait()

    def m2_acc(c, half):
        def gstep(g, cc, c=c, half=half):
            row = half * GR + g * NS2
            for cb in range(D // 16):
                o = cb * 16
                acc = fbuf[row, pl.ds(o, 16)]
                for j in range(1, NS2):
                    acc = acc + fbuf[row + j, pl.ds(o, 16)]
                mbuf[half * GC + g, pl.ds(o, 16)] = acc * inv_ns2
            return cc

        lax.fori_loop(0, GC, gstep, 0)

    def m2_out(c, half):
        pltpu.async_copy(mbuf.at[pl.ds(half * GC, GC)],
                         m2_h.at[pl.ds(b1 + c * GC, GC)], sem_c)

    def m2_waitout(c):
        pltpu.make_async_copy(mbuf.at[pl.ds(0, GC)],
                              m2_h.at[pl.ds(b1 + c * GC, GC)], sem_c).wait()

    m2_fire(0, sem_a, 0)
    m2_fire(1, sem_b, 1)
    m2_waitg(sem_a)
    m2_acc(0, 0)
    m2_out(0, 0)
    m2_fire(2, sem_a, 0)
    m2_waitg(sem_b)
    m2_acc(1, 1)
    m2_out(1, 1)
    m2_fire(3, sem_b, 1)

    def m2_pair(p, carry):
        c0 = 2 * p
        m2_waitg(sem_a)
        m2_waitout(c0 - 2)
        m2_acc(c0, 0)
        m2_out(c0, 0)
        m2_fire(c0 + 2, sem_a, 0)
        m2_waitg(sem_b)
        m2_waitout(c0 - 1)
        m2_acc(c0 + 1, 1)
        m2_out(c0 + 1, 1)
        m2_fire(c0 + 3, sem_b, 1)
        return carry

    lax.fori_loop(1, MC // 2 - 1, m2_pair, 0)

    m2_waitg(sem_a)
    m2_waitout(MC - 4)
    m2_acc(MC - 2, 0)
    m2_out(MC - 2, 0)
    m2_waitg(sem_b)
    m2_waitout(MC - 3)
    m2_acc(MC - 1, 1)
    m2_out(MC - 1, 1)
    m2_waitout(MC - 2)
    m2_waitout(MC - 1)


_sc_gather = pl.kernel(
    _sc_body,
    out_type=[
        jax.ShapeDtypeStruct((B, D), jnp.float32),
        jax.ShapeDtypeStruct((B * NS1, D), jnp.float32),
        jax.ShapeDtypeStruct((B * NS1, D), jnp.float32),
    ],
    mesh=plsc.VectorSubcoreMesh(core_axis_name="c", subcore_axis_name="s",
                                num_cores=2, num_subcores=16),
    scratch_types=[
        pltpu.VMEM((TI,), jnp.int32),
        pltpu.VMEM((T1,), jnp.int32),
        pltpu.VMEM((T1,), jnp.int32),
        pltpu.VMEM((T1 + 16,), jnp.int32),
        pltpu.VMEM((T2,), jnp.int32),
        pltpu.VMEM((T2,), jnp.int32),
        pltpu.VMEM((T2,), jnp.int32),
        pltpu.VMEM((16 * NS1,), jnp.int32),
        pltpu.VMEM((8 * NS2,), jnp.int32),
        pltpu.VMEM((2 * GR, D), jnp.float32),
        pltpu.VMEM((2 * GC, D), jnp.float32),
        pltpu.VMEM((TI, D), jnp.float32),
        pltpu.SemaphoreType.DMA,
        pltpu.SemaphoreType.DMA,
        pltpu.SemaphoreType.DMA,
        pltpu.SemaphoreType.DMA,
    ],
)


def _tc1_body(x1_ref, m2_ref, wx_ref, bx_ref, wn_ref, bn_ref, agg_ref,
              m1_ref, ma_ref, mb_ref):
    x1b = x1_ref[...]
    m2b = m2_ref[...]
    a = jnp.maximum(
        jnp.dot(x1b, wx_ref[...], preferred_element_type=jnp.float32)
        + bx_ref[...], 0.0)
    bb = jnp.maximum(
        jnp.dot(m2b, wn_ref[...], preferred_element_type=jnp.float32)
        + bn_ref[...], 0.0)
    agg = agg_ref[...]
    m1_ref[...] = jnp.dot(agg, x1b, preferred_element_type=jnp.float32)
    ma_ref[...] = jnp.dot(agg, a, preferred_element_type=jnp.float32)
    mb_ref[...] = jnp.dot(agg, bb, preferred_element_type=jnp.float32)


_ROWS1 = 1600
_tc1 = pl.pallas_call(
    _tc1_body,
    grid=(B * NS1 // _ROWS1,),
    in_specs=[
        pl.BlockSpec((_ROWS1, D), lambda i: (i, 0)),
        pl.BlockSpec((_ROWS1, D), lambda i: (i, 0)),
        pl.BlockSpec((D, H), lambda i: (0, 0)),
        pl.BlockSpec((1, H), lambda i: (0, 0)),
        pl.BlockSpec((D, H), lambda i: (0, 0)),
        pl.BlockSpec((1, H), lambda i: (0, 0)),
        pl.BlockSpec((64, _ROWS1), lambda i: (0, 0)),
    ],
    out_specs=[
        pl.BlockSpec((64, D), lambda i: (i, 0)),
        pl.BlockSpec((64, H), lambda i: (i, 0)),
        pl.BlockSpec((64, H), lambda i: (i, 0)),
    ],
    out_shape=[
        jax.ShapeDtypeStruct((B, D), jnp.float32),
        jax.ShapeDtypeStruct((B, H), jnp.float32),
        jax.ShapeDtypeStruct((B, H), jnp.float32),
    ],
)


def _tc2_body(x0_ref, m1_ref, ma_ref, mb_ref,
              wx1_ref, bx1_ref, wn1_ref, bn1_ref,
              wx2t_ref, wx2b_ref, bx2_ref,
              wn2t_ref, wn2b_ref, bn2_ref,
              wfct_ref, wfcb_ref, bfc_ref, out_ref):
    f32 = jnp.float32

    def mm(x, w):
        return jnp.dot(x, w, preferred_element_type=f32)

    h0a = jnp.maximum(mm(x0_ref[...], wx1_ref[...]) + bx1_ref[...], 0.0)
    h0b = jnp.maximum(mm(m1_ref[...], wn1_ref[...]) + bn1_ref[...], 0.0)
    ga = jnp.maximum(
        mm(h0a, wx2t_ref[...]) + mm(h0b, wx2b_ref[...]) + bx2_ref[...], 0.0)
    gb = jnp.maximum(
        mm(ma_ref[...], wn2t_ref[...]) + mm(mb_ref[...], wn2b_ref[...])
        + bn2_ref[...], 0.0)
    nsq = (jnp.sum(ga * ga, axis=1, keepdims=True)
           + jnp.sum(gb * gb, axis=1, keepdims=True))
    inv = 1.0 / jnp.maximum(jnp.sqrt(nsq), 1e-12)
    out_ref[...] = (mm(ga * inv, wfct_ref[...]) + mm(gb * inv, wfcb_ref[...])
                    + bfc_ref[...])


_tc2 = pl.pallas_call(
    _tc2_body,
    out_shape=jax.ShapeDtypeStruct((B, NCLS), jnp.float32),
)


def kernel(ids, feats, adj, W_x1, b_x1, W_n1, b_n1, W_x2, b_x2, W_n2, b_n2,
           W_fc, b_fc):
    ids = ids.astype(jnp.int32)
    cols1 = jax.random.randint(jax.random.key(100), (B, NS1), 0,
                               MAXDEG).astype(jnp.int32).reshape(-1)
    cols2 = jax.random.randint(jax.random.key(101), (B * NS1, NS2), 0,
                               MAXDEG).astype(jnp.int32).reshape(-1)

    adjf = adj.reshape(-1)

    x0, x1, m2 = _sc_gather(ids, adjf, feats, cols1, cols2, _P25, _P10)

    m1, ma, mb = _tc1(x1, m2, W_x1, b_x1.reshape(1, H), W_n1,
                      b_n1.reshape(1, H), _AGG)

    out = _tc2(x0, m1, ma, mb,
               W_x1, b_x1.reshape(1, H), W_n1, b_n1.reshape(1, H),
               W_x2[:H], W_x2[H:], b_x2.reshape(1, H),
               W_n2[:H], W_n2[H:], b_n2.reshape(1, H),
               W_fc[:H], W_fc[H:], b_fc.reshape(1, NCLS))
    return out
